# Initial kernel scaffold; baseline (speedup 1.0000x reference)
import functools

import jax
import jax.numpy as jnp
from jax import lax
from jax.experimental import pallas as pl
from jax.experimental.pallas import tpu as pltpu
from jax.experimental.pallas import tpu_sc as plsc

B, E, N, F = 2, 160000, 10000, 128
NC, NS, L = 2, 16, 16
EPT = E // NS
CH = 80
NCHUNK = EPT // CH
RPT = N // NS
ZR = 125


def _body(msg_hbm, idx_hbm, out_hbm, acc, zbuf, idx_buf, msg_buf):
    b = lax.axis_index("c")
    s = lax.axis_index("s")

    def zrow(i, carry):
        for c0 in range(F // L):
            zbuf[i, pl.ds(c0 * L, L)] = jnp.zeros((L,), jnp.float32)
        return carry

    lax.fori_loop(0, ZR, zrow, 0)
    for j in range(RPT // ZR):
        pltpu.sync_copy(zbuf, acc.at[pl.ds(s * RPT + j * ZR, ZR)])
    plsc.subcore_barrier()

    base = b * E + s * EPT

    def chunk(c, carry):
        off = base + c * CH
        pltpu.sync_copy(idx_hbm.at[pl.ds(off, CH)], idx_buf.at[0])
        pltpu.sync_copy(msg_hbm.at[pl.ds(off, CH)], msg_buf.at[0])
        pltpu.sync_copy(msg_buf.at[0], acc.at[idx_buf.at[0]], add=True)
        return carry

    lax.fori_loop(0, NCHUNK, chunk, 0)
    plsc.subcore_barrier()

    pltpu.sync_copy(
        acc.at[pl.ds(s * RPT, RPT)],
        out_hbm.at[pl.ds(b * N + s * RPT, RPT)],
    )


_scatter_add = functools.partial(
    pl.kernel,
    out_type=jax.ShapeDtypeStruct((B * N, F), jnp.float32),
    mesh=plsc.VectorSubcoreMesh(core_axis_name="c", subcore_axis_name="s"),
    scratch_types=[
        pltpu.VMEM_SHARED((N, F), jnp.float32),
        pltpu.VMEM((ZR, F), jnp.float32),
        pltpu.VMEM((1, CH), jnp.int32),
        pltpu.VMEM((1, CH, F), jnp.float32),
    ],
)(_body)


def kernel(msg_vectors, start_indices, h_v):
    del h_v
    msg2 = msg_vectors.reshape(B * E, F)
    idx1 = start_indices.reshape(B * E).astype(jnp.int32)
    out = _scatter_add(msg2, idx1)
    return out.reshape(B, N, F)


# speedup vs baseline: 4.1976x; 4.1976x over previous
import functools

import jax
import jax.numpy as jnp
from jax import lax
from jax.experimental import pallas as pl
from jax.experimental.pallas import tpu as pltpu
from jax.experimental.pallas import tpu_sc as plsc

B, E, N, F = 2, 160000, 10000, 128
NC, NS, L = 2, 16, 16
EPT = E // NS
CH = 80
NCHUNK = EPT // CH
BLK = 80
NBLK = N // BLK
BPT = -(-NBLK // NS)


def _body(msg_hbm, idx_hbm, out_hbm, acc, zbuf, idx_buf, msg_buf):
    b = lax.axis_index("c")
    s = lax.axis_index("s")

    def zrow(i, carry):
        for c0 in range(F // L):
            zbuf[i, pl.ds(c0 * L, L)] = jnp.zeros((L,), jnp.float32)
        return carry

    lax.fori_loop(0, BLK, zrow, 0)

    def zblk(k, carry):
        blk = s + k * NS

        @pl.when(blk < NBLK)
        def _():
            pltpu.sync_copy(zbuf, acc.at[pl.ds(blk * BLK, BLK)])

        return carry

    lax.fori_loop(0, BPT, zblk, 0)
    plsc.subcore_barrier()

    base = b * E + s * EPT

    def chunk(c, carry):
        off = base + c * CH
        pltpu.sync_copy(idx_hbm.at[pl.ds(off, CH)], idx_buf.at[0])
        pltpu.sync_copy(msg_hbm.at[pl.ds(off, CH)], msg_buf.at[0])
        pltpu.sync_copy(msg_buf.at[0], acc.at[idx_buf.at[0]], add=True)
        return carry

    lax.fori_loop(0, NCHUNK, chunk, 0)
    plsc.subcore_barrier()

    def fblk(k, carry):
        blk = s + k * NS

        @pl.when(blk < NBLK)
        def _():
            pltpu.sync_copy(
                acc.at[pl.ds(blk * BLK, BLK)],
                out_hbm.at[pl.ds(b * N + blk * BLK, BLK)],
            )

        return carry

    lax.fori_loop(0, BPT, fblk, 0)


_scatter_add = functools.partial(
    pl.kernel,
    out_type=jax.ShapeDtypeStruct((B * N, F), jnp.float32),
    mesh=plsc.VectorSubcoreMesh(core_axis_name="c", subcore_axis_name="s"),
    scratch_types=[
        pltpu.VMEM_SHARED((N, F), jnp.float32),
        pltpu.VMEM((BLK, F), jnp.float32),
        pltpu.VMEM((1, CH), jnp.int32),
        pltpu.VMEM((1, CH, F), jnp.float32),
    ],
)(_body)


def kernel(msg_vectors, start_indices, h_v):
    del h_v
    msg2 = msg_vectors.reshape(B * E, F)
    idx1 = start_indices.reshape(B * E).astype(jnp.int32)
    out = _scatter_add(msg2, idx1)
    return out.reshape(B, N, F)


# speedup vs baseline: 10.6724x; 2.5425x over previous
import functools

import jax
import jax.numpy as jnp
from jax import lax
from jax.experimental import pallas as pl
from jax.experimental.pallas import tpu as pltpu
from jax.experimental.pallas import tpu_sc as plsc

B, E, N, F = 2, 160000, 10000, 128
NC, NS, L = 2, 16, 16
EPT = E // NS
CH = 40
NCHUNK = EPT // CH
NB = 5
NGRP = NCHUNK // NB
BLK = 80
NBLK = N // BLK
BPT = -(-NBLK // NS)


def _body(msg_hbm, idx_hbm, out_hbm, acc, zbuf, idx_buf, msg_buf,
          sem_flush, sem_idx, sem_load, sem_add):
    b = lax.axis_index("c")
    s = lax.axis_index("s")
    base = b * E + s * EPT

    for j in range(NB):
        pltpu.async_copy(
            idx_hbm.at[pl.ds(base + j * CH, CH)], idx_buf.at[j, 0], sem_idx[j]
        )
        pltpu.async_copy(
            msg_hbm.at[pl.ds(base + j * CH, CH)], msg_buf.at[j], sem_load[j]
        )

    def zrow(i, carry):
        for c0 in range(F // L):
            zbuf[i, pl.ds(c0 * L, L)] = jnp.zeros((L,), jnp.float32)
        return carry

    lax.fori_loop(0, BLK, zrow, 0)

    def zblk(k, carry):
        blk = s + k * NS

        @pl.when(blk < NBLK)
        def _():
            pltpu.sync_copy(zbuf, acc.at[pl.ds(blk * BLK, BLK)])

        return carry

    lax.fori_loop(0, BPT, zblk, 0)
    plsc.subcore_barrier()

    def group(g, carry):
        for j in range(NB):
            off = base + (g * NB + j) * CH
            pltpu.make_async_copy(
                idx_hbm.at[pl.ds(off, CH)], idx_buf.at[j, 0], sem_idx[j]
            ).wait()
            pltpu.make_async_copy(
                msg_hbm.at[pl.ds(off, CH)], msg_buf.at[j], sem_load[j]
            ).wait()
            pltpu.async_copy(
                msg_buf.at[j], acc.at[idx_buf.at[j, 0]], sem_add[j], add=True
            ).wait()
            pltpu.async_copy(
                idx_hbm.at[pl.ds(off + NB * CH, CH)], idx_buf.at[j, 0],
                sem_idx[j],
            )
            pltpu.async_copy(
                msg_hbm.at[pl.ds(off + NB * CH, CH)], msg_buf.at[j], sem_load[j]
            )
        return carry

    lax.fori_loop(0, NGRP - 1, group, 0)

    for j in range(NB):
        off = base + ((NGRP - 1) * NB + j) * CH
        pltpu.make_async_copy(
            idx_hbm.at[pl.ds(off, CH)], idx_buf.at[j, 0], sem_idx[j]
        ).wait()
        pltpu.make_async_copy(
            msg_hbm.at[pl.ds(off, CH)], msg_buf.at[j], sem_load[j]
        ).wait()
        pltpu.async_copy(
            msg_buf.at[j], acc.at[idx_buf.at[j, 0]], sem_add[j], add=True
        ).wait()

    plsc.subcore_barrier()

    nf = 0
    for k in range(BPT):
        blk = s + k * NS

        @pl.when(blk < NBLK)
        def _():
            pltpu.async_copy(
                acc.at[pl.ds(blk * BLK, BLK)],
                out_hbm.at[pl.ds(b * N + blk * BLK, BLK)],
                sem_flush,
            )

    def fdrain(k, carry):
        blk = s + k * NS

        @pl.when(blk < NBLK)
        def _():
            pltpu.make_async_copy(
                acc.at[pl.ds(0, BLK)], out_hbm.at[pl.ds(0, BLK)], sem_flush
            ).wait()

        return carry

    lax.fori_loop(0, BPT, fdrain, 0)


_scatter_add = functools.partial(
    pl.kernel,
    out_type=jax.ShapeDtypeStruct((B * N, F), jnp.float32),
    mesh=plsc.VectorSubcoreMesh(core_axis_name="c", subcore_axis_name="s"),
    scratch_types=[
        pltpu.VMEM_SHARED((N, F), jnp.float32),
        pltpu.VMEM((BLK, F), jnp.float32),
        pltpu.VMEM((NB, 1, CH), jnp.int32),
        pltpu.VMEM((NB, CH, F), jnp.float32),
        pltpu.SemaphoreType.DMA,
        [pltpu.SemaphoreType.DMA] * NB,
        [pltpu.SemaphoreType.DMA] * NB,
        [pltpu.SemaphoreType.DMA] * NB,
    ],
)(_body)


def kernel(msg_vectors, start_indices, h_v):
    del h_v
    msg2 = msg_vectors.reshape(B * E, F)
    idx1 = start_indices.reshape(B * E).astype(jnp.int32)
    out = _scatter_add(msg2, idx1)
    return out.reshape(B, N, F)
